# Initial kernel scaffold; baseline (speedup 1.0000x reference)
import jax
import jax.numpy as jnp
from jax.experimental import pallas as pl


def kernel(x, edge_index, W_e1, b_e1, W_e2, b_e2, W_a1, b_a1, W_a2, b_a2, W_s1, b_s1):
    raise NotImplementedError("write your pallas kernel here")


# speedup vs baseline: 12.1562x; 12.1562x over previous
import functools

import jax
import jax.numpy as jnp
from jax import lax
from jax.experimental import pallas as pl
from jax.experimental.pallas import tpu as pltpu
from jax.experimental.pallas import tpu_sc as plsc

NC = 2
NS = 16
NW = NC * NS
CHUNK = 128
N_ACC = 10112
ROWS_PER_TILE = N_ACC // NS


def _sc_scatter_stage(table, src3, dst3, zeros_acc, n, width, n_chunks,
                      do_gather, ones_rows=None):
  mesh = plsc.VectorSubcoreMesh(core_axis_name="c", subcore_axis_name="s",
                                num_cores=NC, num_subcores=NS)

  scratch = [
      pltpu.VMEM((n_chunks, CHUNK), jnp.int32),
      pltpu.VMEM((n_chunks, CHUNK), jnp.int32),
      pltpu.VMEM((CHUNK, width), jnp.float32),
      pltpu.VMEM_SHARED((N_ACC, width), jnp.float32),
      pltpu.SemaphoreType.DMA,
  ]

  @functools.partial(
      pl.kernel,
      out_type=jax.ShapeDtypeStruct((NC * N_ACC, width), jnp.float32),
      mesh=mesh,
      scratch_types=scratch,
      compiler_params=pltpu.CompilerParams(use_tc_tiling_on_sc=False),
  )
  def body(*refs):
    if do_gather:
      table_h, src_h, dst_h, zeros_h, out_h, src_v, dst_v, rows_v, acc_sh, sem = refs
    else:
      ones_h, src_h, dst_h, zeros_h, out_h, src_v, dst_v, rows_v, acc_sh, sem = refs
    c = lax.axis_index("c")
    s = lax.axis_index("s")
    wid = s * NC + c

    pltpu.sync_copy(src_h.at[wid], src_v)
    pltpu.sync_copy(dst_h.at[wid], dst_v)

    pltpu.sync_copy(zeros_h.at[pl.ds(s * ROWS_PER_TILE, ROWS_PER_TILE)],
                    acc_sh.at[pl.ds(s * ROWS_PER_TILE, ROWS_PER_TILE)])
    if not do_gather:
      pltpu.sync_copy(ones_h, rows_v)
    plsc.subcore_barrier()

    def chunk_body(j, carry):
      if do_gather:
        pltpu.async_copy(table_h.at[src_v.at[j]], rows_v, sem).wait()
      pltpu.sync_copy(rows_v, acc_sh.at[dst_v.at[j]], add=True)
      return carry

    lax.fori_loop(0, n_chunks, chunk_body, 0)
    plsc.subcore_barrier()

    base = s * ROWS_PER_TILE
    pltpu.sync_copy(acc_sh.at[pl.ds(base, ROWS_PER_TILE)],
                    out_h.at[pl.ds(c * N_ACC + base, ROWS_PER_TILE)])

  if do_gather:
    return body(table, src3, dst3, zeros_acc)
  else:
    return body(ones_rows, src3, dst3, zeros_acc)


def _split_partials(out2, n):
  return out2[:n], out2[N_ACC:N_ACC + n]


RB = 2000


def _row_specs(shapes):
  specs = []
  for kind, shp in shapes:
    if kind == "row":
      specs.append(pl.BlockSpec((RB, shp), lambda i: (i, 0)))
    else:
      specs.append(pl.BlockSpec(shp, lambda i, r=len(shp): (0,) * r))
  return specs


def _tc_call(fn, in_shapes, out_shapes, n, args):
  grid = (n // RB,)
  return pl.pallas_call(
      fn,
      grid=grid,
      in_specs=_row_specs(in_shapes),
      out_specs=_row_specs(out_shapes),
      out_shape=[jax.ShapeDtypeStruct((n, k), jnp.float32)
                 for _, k in out_shapes],
  )(*args)


def _tc1(dA, dB, x, W, dinv_o, t1_o, ys1_o):
  deg = dA[:, :1] + dB[:, :1] + 1.0
  dinv = lax.rsqrt(deg)
  t1 = jnp.dot(x[...], W[...], preferred_element_type=jnp.float32)
  dinv_o[...] = dinv
  t1_o[...] = t1
  ys1_o[...] = dinv * t1


def _tc_mid(relu, t_p, Sa, Sb, dinv, Wn, b, t_o, ys_o, h_o=None):
  dv = dinv[...]
  u = dv * (Sa[...] + Sb[...]) + dv * dv * t_p[...] + b[...]
  if relu:
    u = jnp.maximum(u, 0.0)
  if h_o is not None:
    h_o[...] = u
  t = jnp.dot(u, Wn[...], preferred_element_type=jnp.float32)
  t_o[...] = t
  ys_o[...] = dv * t


def _tc4(t3, Sa, Sb, dinv, b, h, x1_o, ys45_o):
  dv = dinv[...]
  x1 = jnp.maximum(dv * (Sa[...] + Sb[...]) + dv * dv * t3[...] + b[...], 0.0)
  x1_o[...] = x1
  ys45_o[...] = jnp.concatenate([dv * x1, dv * h[...]], axis=1)


def _tc5(Sa, Sb, dinv, x1, h, Wa2, ba2, Ws1, bs1, x_o, h_o):
  dv = dinv[...]
  S = Sa[...] + Sb[...]
  A4 = dv * S[:, :64] + dv * dv * x1[...]
  A5 = dv * S[:, 64:] + dv * dv * h[...]
  x_o[...] = jnp.dot(A4, Wa2[...], preferred_element_type=jnp.float32) + ba2[...]
  h_o[...] = jnp.dot(A5, Ws1[...], preferred_element_type=jnp.float32) + bs1[...]


MB_R = 1000
MB_C = 2048


def _adj_kernel(hi, hTj, o):
  o[...] = jnp.dot(hi[...], hTj[...], preferred_element_type=jnp.float32)


def kernel(x, edge_index, W_e1, b_e1, W_e2, b_e2, W_a1, b_a1, W_a2, b_a2,
           W_s1, b_s1):
  n = x.shape[0]
  e = edge_index.shape[1]

  n_chunks = -(-e // (NW * CHUNK))
  e_pad = n_chunks * NW * CHUNK
  src = jnp.concatenate(
      [edge_index[0], jnp.zeros((e_pad - e,), jnp.int32)])
  dst = jnp.concatenate(
      [edge_index[1], jnp.full((e_pad - e,), n, jnp.int32)])
  src3 = src.reshape(NW, n_chunks, CHUNK)
  dst3 = dst.reshape(NW, n_chunks, CHUNK)

  zeros64 = jnp.zeros((N_ACC, 64), jnp.float32)
  zeros128 = jnp.zeros((N_ACC, 128), jnp.float32)
  zeros16 = jnp.zeros((N_ACC, 16), jnp.float32)
  ones16 = jnp.ones((CHUNK, 16), jnp.float32)

  stage = functools.partial(_sc_scatter_stage, n=n, n_chunks=n_chunks)

  degp = stage(None, src3, dst3, zeros16, width=16, do_gather=False,
               ones_rows=ones16)
  dA, dB = _split_partials(degp, n)

  b_e1r = b_e1.reshape(1, 64)
  b_e2r = b_e2.reshape(1, 64)
  b_a1r = b_a1.reshape(1, 64)
  dinv, t1, ys1 = _tc_call(
      _tc1,
      [("row", 16), ("row", 16), ("row", 128), ("full", (128, 64))],
      [("row", 1), ("row", 64), ("row", 64)],
      n, (dA, dB, x, W_e1))

  s1 = stage(ys1, src3, dst3, zeros64, width=64, do_gather=True)
  S1a, S1b = _split_partials(s1, n)

  t2, ys2 = _tc_call(
      functools.partial(_tc_mid, True),
      [("row", 64), ("row", 64), ("row", 64), ("row", 1),
       ("full", (64, 64)), ("full", (1, 64))],
      [("row", 64), ("row", 64)],
      n, (t1, S1a, S1b, dinv, W_e2, b_e1r))

  s2 = stage(ys2, src3, dst3, zeros64, width=64, do_gather=True)
  S2a, S2b = _split_partials(s2, n)

  def _tc3(t_p, Sa, Sb, dinv, Wn, b, t_o, ys_o, h_o):
    _tc_mid(False, t_p, Sa, Sb, dinv, Wn, b, t_o, ys_o, h_o)

  t3, ys3, h = _tc_call(
      _tc3,
      [("row", 64), ("row", 64), ("row", 64), ("row", 1),
       ("full", (64, 64)), ("full", (1, 64))],
      [("row", 64), ("row", 64), ("row", 64)],
      n, (t2, S2a, S2b, dinv, W_a1, b_e2r))

  s3 = stage(ys3, src3, dst3, zeros64, width=64, do_gather=True)
  S3a, S3b = _split_partials(s3, n)

  x1, ys45 = _tc_call(
      _tc4,
      [("row", 64), ("row", 64), ("row", 64), ("row", 1),
       ("full", (1, 64)), ("row", 64)],
      [("row", 64), ("row", 128)],
      n, (t3, S3a, S3b, dinv, b_a1r, h))

  s45 = stage(ys45, src3, dst3, zeros128, width=128, do_gather=True)
  S45a, S45b = _split_partials(s45, n)

  x_, h_ = _tc_call(
      _tc5,
      [("row", 128), ("row", 128), ("row", 1), ("row", 64), ("row", 64),
       ("full", (64, 128)), ("full", (1, 128)),
       ("full", (64, 128)), ("full", (1, 128))],
      [("row", 128), ("row", 128)],
      n, (S45a, S45b, dinv, x1, h, W_a2, b_a2.reshape(1, 128),
          W_s1, b_s1.reshape(1, 128)))

  hT = h_.T
  adj_ = pl.pallas_call(
      _adj_kernel,
      grid=(n // MB_R, pl.cdiv(n, MB_C)),
      in_specs=[
          pl.BlockSpec((MB_R, 128), lambda i, j: (i, 0)),
          pl.BlockSpec((128, MB_C), lambda i, j: (0, j)),
      ],
      out_specs=pl.BlockSpec((MB_R, MB_C), lambda i, j: (i, j)),
      out_shape=jax.ShapeDtypeStruct((n, n), jnp.float32),
  )(h_, hT)

  return (x_, adj_)
